# Initial kernel scaffold; baseline (speedup 1.0000x reference)
import functools

import jax
import jax.numpy as jnp
from jax import lax
from jax.experimental import pallas as pl
from jax.experimental.pallas import tpu as pltpu
from jax.experimental.pallas import tpu_sc as plsc

_N_NODES = 10000
_D = 128
_E = 320000

_NC = 2
_NS = 16
_NW = _NC * _NS
_EPW = _E // _NW
_C = 80
_NCH = _EPW // _C
_G = _C // 16


def _make_sc_kernel():
    mesh = plsc.VectorSubcoreMesh(core_axis_name="c", subcore_axis_name="s")

    @functools.partial(
        pl.kernel,
        mesh=mesh,
        out_type=jax.ShapeDtypeStruct((_E,), jnp.float32),
        scratch_types=[
            pltpu.VMEM((_EPW,), jnp.int32),
            pltpu.VMEM((_EPW,), jnp.int32),
            pltpu.VMEM((_C, _D), jnp.float32),
            pltpu.VMEM((_C, _D), jnp.float32),
            pltpu.VMEM((_EPW,), jnp.float32),
            pltpu.SemaphoreType.DMA,
            pltpu.SemaphoreType.DMA,
        ],
    )
    def edge_dot(src_hbm, tgt_hbm, edge_hbm, out_hbm,
                 idx_s, idx_t, rows_s, rows_t, outbuf, sem_s, sem_t):
        wid = lax.axis_index("s") * _NC + lax.axis_index("c")
        base = wid * _EPW
        pltpu.sync_copy(edge_hbm.at[0, pl.ds(base, _EPW)], idx_s)
        pltpu.sync_copy(edge_hbm.at[1, pl.ds(base, _EPW)], idx_t)

        lane = lax.iota(jnp.int32, 16)

        def chunk_body(i, carry):
            off = i * _C
            cp_s = pltpu.async_copy(
                src_hbm.at[idx_s.at[pl.ds(off, _C)]], rows_s, sem_s)
            cp_t = pltpu.async_copy(
                tgt_hbm.at[idx_t.at[pl.ds(off, _C)]], rows_t, sem_t)
            cp_s.wait()
            cp_t.wait()
            for g in range(_G):
                row_ids = jnp.int32(g * 16) + lane
                accs = [jnp.zeros((16,), jnp.float32) for _ in range(4)]
                col = jnp.zeros((16,), jnp.int32)
                for d in range(_D):
                    vs = plsc.load_gather(rows_s, [row_ids, col])
                    vt = plsc.load_gather(rows_t, [row_ids, col])
                    accs[d % 4] = accs[d % 4] + vs * vt
                    col = col + 1
                tot = (accs[0] + accs[1]) + (accs[2] + accs[3])
                outbuf[pl.ds(off + g * 16, 16)] = tot
            return carry

        lax.fori_loop(0, _NCH, chunk_body, 0)
        pltpu.sync_copy(outbuf, out_hbm.at[pl.ds(base, _EPW)])

    return edge_dot


_sc_kernel = _make_sc_kernel()


def kernel(source_node_emb, target_node_emb, edge_label_index):
    return _sc_kernel(source_node_emb, target_node_emb, edge_label_index)


# speedup vs baseline: 2.7235x; 2.7235x over previous
import functools

import jax
import jax.numpy as jnp
from jax import lax
from jax.experimental import pallas as pl
from jax.experimental.pallas import tpu as pltpu
from jax.experimental.pallas import tpu_sc as plsc

_N_NODES = 10000
_D = 128
_E = 320000

_NC = 2
_NS = 16
_NW = _NC * _NS
_EPW = _E // _NW
_C = 80
_NCH = _EPW // _C
_G = _C // 16


def _make_sc_kernel():
    mesh = plsc.VectorSubcoreMesh(core_axis_name="c", subcore_axis_name="s")

    @functools.partial(
        pl.kernel,
        mesh=mesh,
        compiler_params=pltpu.CompilerParams(needs_layout_passes=False),
        out_type=jax.ShapeDtypeStruct((_E,), jnp.float32),
        scratch_types=[
            pltpu.VMEM((_EPW,), jnp.int32),
            pltpu.VMEM((_EPW,), jnp.int32),
            pltpu.VMEM((_C, _D), jnp.float32),
            pltpu.VMEM((_C, _D), jnp.float32),
            pltpu.VMEM((_EPW,), jnp.float32),
            pltpu.SemaphoreType.DMA,
            pltpu.SemaphoreType.DMA,
        ],
    )
    def edge_dot(src_hbm, tgt_hbm, edge_hbm, out_hbm,
                 idx_s, idx_t, rows_s, rows_t, outbuf, sem_s, sem_t):
        wid = lax.axis_index("s") * _NC + lax.axis_index("c")
        base = wid * _EPW
        pltpu.sync_copy(edge_hbm.at[pl.ds(base, _EPW)], idx_s)
        pltpu.sync_copy(edge_hbm.at[pl.ds(_E + base, _EPW)], idx_t)

        lane = lax.iota(jnp.int32, 16)
        last_lane = lane == jnp.int32(15)

        def chunk_body(i, carry):
            off = i * _C
            cp_s = pltpu.async_copy(
                src_hbm.at[idx_s.at[pl.ds(off, _C)]], rows_s, sem_s)
            cp_t = pltpu.async_copy(
                tgt_hbm.at[idx_t.at[pl.ds(off, _C)]], rows_t, sem_t)
            cp_s.wait()
            cp_t.wait()
            for e in range(_C):
                acc = rows_s[e, pl.ds(0, 16)] * rows_t[e, pl.ds(0, 16)]
                for k in range(1, _D // 16):
                    acc = acc + (rows_s[e, pl.ds(16 * k, 16)]
                                 * rows_t[e, pl.ds(16 * k, 16)])
                tot = plsc.cumsum(acc)
                eidx = jnp.broadcast_to(off + jnp.int32(e), (16,))
                plsc.store_scatter(outbuf, [eidx], tot, mask=last_lane)
            return carry

        lax.fori_loop(0, _NCH, chunk_body, 0)
        pltpu.sync_copy(outbuf, out_hbm.at[pl.ds(base, _EPW)])

    return edge_dot


_sc_kernel = _make_sc_kernel()


def kernel(source_node_emb, target_node_emb, edge_label_index):
    return _sc_kernel(source_node_emb, target_node_emb,
                      edge_label_index.reshape(-1))


# speedup vs baseline: 3.4538x; 1.2681x over previous
import functools

import jax
import jax.numpy as jnp
from jax import lax
from jax.experimental import pallas as pl
from jax.experimental.pallas import tpu as pltpu
from jax.experimental.pallas import tpu_sc as plsc

_N_NODES = 10000
_D = 128
_E = 320000

_NC = 2
_NS = 16
_NW = _NC * _NS
_EPW = _E // _NW
_C = 80
_NCH = _EPW // _C
_G = _C // 16


def _make_sc_kernel():
    mesh = plsc.VectorSubcoreMesh(core_axis_name="c", subcore_axis_name="s")

    @functools.partial(
        pl.kernel,
        mesh=mesh,
        compiler_params=pltpu.CompilerParams(needs_layout_passes=False),
        out_type=jax.ShapeDtypeStruct((_E,), jnp.float32),
        scratch_types=[
            pltpu.VMEM((_EPW,), jnp.int32),
            pltpu.VMEM((_EPW,), jnp.int32),
            pltpu.VMEM((_C, _D), jnp.float32),
            pltpu.VMEM((_C, _D), jnp.float32),
            pltpu.VMEM((_C, _D), jnp.float32),
            pltpu.VMEM((_C, _D), jnp.float32),
            pltpu.VMEM((_EPW,), jnp.float32),
            pltpu.SemaphoreType.DMA,
            pltpu.SemaphoreType.DMA,
            pltpu.SemaphoreType.DMA,
            pltpu.SemaphoreType.DMA,
        ],
    )
    def edge_dot(src_hbm, tgt_hbm, edge_hbm, out_hbm,
                 idx_s, idx_t, rows_s0, rows_s1, rows_t0, rows_t1, outbuf,
                 sem_s0, sem_s1, sem_t0, sem_t1):
        wid = lax.axis_index("s") * _NC + lax.axis_index("c")
        base = wid * _EPW
        pltpu.sync_copy(edge_hbm.at[pl.ds(base, _EPW)], idx_s)
        pltpu.sync_copy(edge_hbm.at[pl.ds(_E + base, _EPW)], idx_t)

        lane = lax.iota(jnp.int32, 16)
        last_lane = lane == jnp.int32(15)
        rows_s = (rows_s0, rows_s1)
        rows_t = (rows_t0, rows_t1)
        sem_s = (sem_s0, sem_s1)
        sem_t = (sem_t0, sem_t1)

        def start(j, p):
            off = j * _C
            pltpu.async_copy(
                src_hbm.at[idx_s.at[pl.ds(off, _C)]], rows_s[p], sem_s[p])
            pltpu.async_copy(
                tgt_hbm.at[idx_t.at[pl.ds(off, _C)]], rows_t[p], sem_t[p])

        def wait(p):
            pltpu.make_async_copy(
                src_hbm.at[idx_s.at[pl.ds(0, _C)]], rows_s[p], sem_s[p]
            ).wait()
            pltpu.make_async_copy(
                tgt_hbm.at[idx_t.at[pl.ds(0, _C)]], rows_t[p], sem_t[p]
            ).wait()

        def compute(j, p):
            off = j * _C
            rs, rt = rows_s[p], rows_t[p]
            for e in range(_C):
                acc = rs[e, pl.ds(0, 16)] * rt[e, pl.ds(0, 16)]
                for k in range(1, _D // 16):
                    acc = acc + (rs[e, pl.ds(16 * k, 16)]
                                 * rt[e, pl.ds(16 * k, 16)])
                tot = plsc.cumsum(acc)
                eidx = jnp.broadcast_to(off + jnp.int32(e), (16,))
                plsc.store_scatter(outbuf, [eidx], tot, mask=last_lane)

        start(0, 0)
        start(1, 1)

        def pair_body(k, carry):
            j0 = 2 * k
            wait(0)
            compute(j0, 0)
            pl.when(j0 + 2 < _NCH)(lambda: start(j0 + 2, 0))
            wait(1)
            compute(j0 + 1, 1)
            pl.when(j0 + 3 < _NCH)(lambda: start(j0 + 3, 1))
            return carry

        lax.fori_loop(0, (_NCH - 1) // 2, pair_body, 0)
        wait(0)
        compute(_NCH - 1, 0)
        pltpu.sync_copy(outbuf, out_hbm.at[pl.ds(base, _EPW)])

    return edge_dot


_sc_kernel = _make_sc_kernel()


def kernel(source_node_emb, target_node_emb, edge_label_index):
    return _sc_kernel(source_node_emb, target_node_emb,
                      edge_label_index.reshape(-1))


# speedup vs baseline: 3.9768x; 1.1514x over previous
import functools

import jax
import jax.numpy as jnp
from jax import lax
from jax.experimental import pallas as pl
from jax.experimental.pallas import tpu as pltpu
from jax.experimental.pallas import tpu_sc as plsc

_N_NODES = 10000
_D = 128
_E = 320000

_NC = 2
_NS = 16
_NW = _NC * _NS
_EPW = _E // _NW
_C = 80
_NCH = _EPW // _C
_G = _C // 16


def _make_sc_kernel():
    mesh = plsc.VectorSubcoreMesh(core_axis_name="c", subcore_axis_name="s")

    @functools.partial(
        pl.kernel,
        mesh=mesh,
        compiler_params=pltpu.CompilerParams(needs_layout_passes=False),
        out_type=jax.ShapeDtypeStruct((_E,), jnp.float32),
        scratch_types=[
            pltpu.VMEM((_EPW,), jnp.int32),
            pltpu.VMEM((_EPW,), jnp.int32),
            pltpu.VMEM((_C, _D), jnp.float32),
            pltpu.VMEM((_C, _D), jnp.float32),
            pltpu.VMEM((_C, _D), jnp.float32),
            pltpu.VMEM((_C, _D), jnp.float32),
            pltpu.VMEM((_EPW,), jnp.float32),
            pltpu.SemaphoreType.DMA,
            pltpu.SemaphoreType.DMA,
            pltpu.SemaphoreType.DMA,
            pltpu.SemaphoreType.DMA,
        ],
    )
    def edge_dot(src_hbm, tgt_hbm, edge_hbm, out_hbm,
                 idx_s, idx_t, rows_s0, rows_s1, rows_t0, rows_t1, outbuf,
                 sem_s0, sem_s1, sem_t0, sem_t1):
        wid = lax.axis_index("s") * _NC + lax.axis_index("c")
        base = wid * _EPW
        pltpu.sync_copy(edge_hbm.at[pl.ds(base, _EPW)], idx_s)
        pltpu.sync_copy(edge_hbm.at[pl.ds(_E + base, _EPW)], idx_t)

        lane = lax.iota(jnp.int32, 16)
        masks = [(lane & (1 << k)) == 0 for k in range(4)]
        xors = [lane ^ (1 << k) for k in range(4)]

        def lane_permute(v, idx):
            return lax.gather(
                v, idx.reshape(16, 1),
                lax.GatherDimensionNumbers(
                    offset_dims=(), collapsed_slice_dims=(0,),
                    start_index_map=(0,)),
                slice_sizes=(1,),
                mode=lax.GatherScatterMode.PROMISE_IN_BOUNDS)
        rows_s = (rows_s0, rows_s1)
        rows_t = (rows_t0, rows_t1)
        sem_s = (sem_s0, sem_s1)
        sem_t = (sem_t0, sem_t1)

        def start(j, p):
            off = j * _C
            pltpu.async_copy(
                src_hbm.at[idx_s.at[pl.ds(off, _C)]], rows_s[p], sem_s[p])
            pltpu.async_copy(
                tgt_hbm.at[idx_t.at[pl.ds(off, _C)]], rows_t[p], sem_t[p])

        def wait(p):
            pltpu.make_async_copy(
                src_hbm.at[idx_s.at[pl.ds(0, _C)]], rows_s[p], sem_s[p]
            ).wait()
            pltpu.make_async_copy(
                tgt_hbm.at[idx_t.at[pl.ds(0, _C)]], rows_t[p], sem_t[p]
            ).wait()

        def compute(j, p):
            off = j * _C
            rs, rt = rows_s[p], rows_t[p]
            def combine(a, b, k):
                m, xi = masks[k], xors[k]
                return (jnp.where(m, a, b)
                        + lane_permute(jnp.where(m, b, a), xi))

            for g in range(_C // 16):
                bufs = [None] * 5
                for jj in range(16):
                    e = g * 16 + jj
                    a0 = rs[e, pl.ds(0, 16)] * rt[e, pl.ds(0, 16)]
                    a1 = rs[e, pl.ds(16, 16)] * rt[e, pl.ds(16, 16)]
                    for k in range(2, _D // 16, 2):
                        a0 = a0 + (rs[e, pl.ds(16 * k, 16)]
                                   * rt[e, pl.ds(16 * k, 16)])
                        a1 = a1 + (rs[e, pl.ds(16 * (k + 1), 16)]
                                   * rt[e, pl.ds(16 * (k + 1), 16)])
                    v = a0 + a1
                    k = 0
                    while bufs[k] is not None:
                        v = combine(bufs[k], v, k)
                        bufs[k] = None
                        k += 1
                    bufs[k] = v
                outbuf[pl.ds(off + g * 16, 16)] = bufs[4]

        start(0, 0)
        start(1, 1)

        def pair_body(k, carry):
            j0 = 2 * k
            wait(0)
            compute(j0, 0)
            pl.when(j0 + 2 < _NCH)(lambda: start(j0 + 2, 0))
            wait(1)
            compute(j0 + 1, 1)
            pl.when(j0 + 3 < _NCH)(lambda: start(j0 + 3, 1))
            return carry

        lax.fori_loop(0, (_NCH - 1) // 2, pair_body, 0)
        wait(0)
        compute(_NCH - 1, 0)
        pltpu.sync_copy(outbuf, out_hbm.at[pl.ds(base, _EPW)])

    return edge_dot


_sc_kernel = _make_sc_kernel()


def kernel(source_node_emb, target_node_emb, edge_label_index):
    return _sc_kernel(source_node_emb, target_node_emb,
                      edge_label_index.reshape(-1))


# speedup vs baseline: 8.0704x; 2.0294x over previous
import functools

import jax
import jax.numpy as jnp
from jax import lax
from jax.experimental import pallas as pl
from jax.experimental.pallas import tpu as pltpu
from jax.experimental.pallas import tpu_sc as plsc

_N_NODES = 10000
_D = 128
_E = 320000

_NC = 2
_NS = 16
_NW = _NC * _NS
_EPW = _E // _NW
_C = 80
_NCH = _EPW // _C
_G = _C // 16


def _make_sc_kernel():
    mesh = plsc.VectorSubcoreMesh(core_axis_name="c", subcore_axis_name="s")

    @functools.partial(
        pl.kernel,
        mesh=mesh,
        compiler_params=pltpu.CompilerParams(needs_layout_passes=False),
        out_type=jax.ShapeDtypeStruct((_E,), jnp.float32),
        scratch_types=[
            pltpu.VMEM((_EPW,), jnp.int32),
            pltpu.VMEM((_EPW,), jnp.int32),
            pltpu.VMEM((_C, _D), jnp.float32),
            pltpu.VMEM((_C, _D), jnp.float32),
            pltpu.VMEM((_C, _D), jnp.float32),
            pltpu.VMEM((_C, _D), jnp.float32),
            pltpu.VMEM((_EPW,), jnp.float32),
            pltpu.SemaphoreType.DMA,
            pltpu.SemaphoreType.DMA,
            pltpu.SemaphoreType.DMA,
            pltpu.SemaphoreType.DMA,
        ],
    )
    def edge_dot(src_hbm, tgt_hbm, edge_hbm, out_hbm,
                 idx_s, idx_t, rows_s0, rows_s1, rows_t0, rows_t1, outbuf,
                 sem_s0, sem_s1, sem_t0, sem_t1):
        wid = lax.axis_index("s") * _NC + lax.axis_index("c")
        base = wid * _EPW
        pltpu.sync_copy(edge_hbm.at[pl.ds(base, _EPW)], idx_s)
        pltpu.sync_copy(edge_hbm.at[pl.ds(_E + base, _EPW)], idx_t)

        lane = lax.iota(jnp.int32, 16)
        masks = [(lane & (1 << k)) == 0 for k in range(4)]
        xors = [lane ^ (1 << k) for k in range(4)]

        def lane_permute(v, idx):
            return lax.gather(
                v, idx.reshape(16, 1),
                lax.GatherDimensionNumbers(
                    offset_dims=(), collapsed_slice_dims=(0,),
                    start_index_map=(0,)),
                slice_sizes=(1,),
                mode=lax.GatherScatterMode.PROMISE_IN_BOUNDS)
        rows_s = (rows_s0, rows_s1)
        rows_t = (rows_t0, rows_t1)
        sem_s = (sem_s0, sem_s1)
        sem_t = (sem_t0, sem_t1)

        def start(j, p):
            off = j * _C
            pltpu.async_copy(
                src_hbm.at[idx_s.at[pl.ds(off, _C)]], rows_s[p], sem_s[p])
            pltpu.async_copy(
                tgt_hbm.at[idx_t.at[pl.ds(off, _C)]], rows_t[p], sem_t[p])

        def wait(p):
            pltpu.make_async_copy(
                src_hbm.at[idx_s.at[pl.ds(0, _C)]], rows_s[p], sem_s[p]
            ).wait()
            pltpu.make_async_copy(
                tgt_hbm.at[idx_t.at[pl.ds(0, _C)]], rows_t[p], sem_t[p]
            ).wait()

        def compute(j, p):
            off = j * _C
            rs, rt = rows_s[p], rows_t[p]
            for g in range(_C // 16):

                @plsc.parallel_loop(0, 16, unroll=4,
                                    carry=jnp.zeros((16,), jnp.float32))
                def acc_loop(jj, acc, g=g):
                    e = g * 16 + jj
                    a0 = rs[e, pl.ds(0, 16)] * rt[e, pl.ds(0, 16)]
                    a1 = rs[e, pl.ds(16, 16)] * rt[e, pl.ds(16, 16)]
                    for k in range(2, _D // 16, 2):
                        a0 = a0 + (rs[e, pl.ds(16 * k, 16)]
                                   * rt[e, pl.ds(16 * k, 16)])
                        a1 = a1 + (rs[e, pl.ds(16 * (k + 1), 16)]
                                   * rt[e, pl.ds(16 * (k + 1), 16)])
                    v = a0 + a1
                    for k in range(4):
                        v = v + lane_permute(v, xors[k])
                    return jnp.where(lane == jj, v, acc)

                outbuf[pl.ds(off + g * 16, 16)] = acc_loop

        start(0, 0)
        start(1, 1)

        def pair_body(k, carry):
            j0 = 2 * k
            wait(0)
            compute(j0, 0)
            pl.when(j0 + 2 < _NCH)(lambda: start(j0 + 2, 0))
            wait(1)
            compute(j0 + 1, 1)
            pl.when(j0 + 3 < _NCH)(lambda: start(j0 + 3, 1))
            return carry

        lax.fori_loop(0, (_NCH - 1) // 2, pair_body, 0)
        wait(0)
        compute(_NCH - 1, 0)
        pltpu.sync_copy(outbuf, out_hbm.at[pl.ds(base, _EPW)])

    return edge_dot


_sc_kernel = _make_sc_kernel()


def kernel(source_node_emb, target_node_emb, edge_label_index):
    return _sc_kernel(source_node_emb, target_node_emb,
                      edge_label_index.reshape(-1))


# speedup vs baseline: 8.3125x; 1.0300x over previous
import functools

import jax
import jax.numpy as jnp
from jax import lax
from jax.experimental import pallas as pl
from jax.experimental.pallas import tpu as pltpu
from jax.experimental.pallas import tpu_sc as plsc

_N_NODES = 10000
_D = 128
_E = 320000

_NC = 2
_NS = 16
_NW = _NC * _NS
_EPW = _E // _NW
_C = 80
_NCH = _EPW // _C
_NBUF = 4


def _make_sc_kernel():
    mesh = plsc.VectorSubcoreMesh(core_axis_name="c", subcore_axis_name="s")

    @functools.partial(
        pl.kernel,
        mesh=mesh,
        compiler_params=pltpu.CompilerParams(needs_layout_passes=False),
        out_type=jax.ShapeDtypeStruct((_E,), jnp.float32),
        scratch_types=(
            [pltpu.VMEM((_EPW,), jnp.int32)] * 2
            + [pltpu.VMEM((_C, _D), jnp.float32)] * (2 * _NBUF)
            + [pltpu.VMEM((_EPW,), jnp.float32)]
            + [pltpu.SemaphoreType.DMA] * (2 * _NBUF)
        ),
    )
    def edge_dot(src_hbm, tgt_hbm, edge_hbm, out_hbm,
                 idx_s, idx_t,
                 rows_s0, rows_s1, rows_s2, rows_s3,
                 rows_t0, rows_t1, rows_t2, rows_t3, outbuf,
                 sem_s0, sem_s1, sem_s2, sem_s3,
                 sem_t0, sem_t1, sem_t2, sem_t3):
        wid = lax.axis_index("s") * _NC + lax.axis_index("c")
        base = wid * _EPW
        pltpu.sync_copy(edge_hbm.at[pl.ds(base, _EPW)], idx_s)
        pltpu.sync_copy(edge_hbm.at[pl.ds(_E + base, _EPW)], idx_t)

        lane = lax.iota(jnp.int32, 16)
        xors = [lane ^ (1 << k) for k in range(4)]
        rows_s = (rows_s0, rows_s1, rows_s2, rows_s3)
        rows_t = (rows_t0, rows_t1, rows_t2, rows_t3)
        sem_s = (sem_s0, sem_s1, sem_s2, sem_s3)
        sem_t = (sem_t0, sem_t1, sem_t2, sem_t3)

        def lane_permute(v, idx):
            return lax.gather(
                v, idx.reshape(16, 1),
                lax.GatherDimensionNumbers(
                    offset_dims=(), collapsed_slice_dims=(0,),
                    start_index_map=(0,)),
                slice_sizes=(1,),
                mode=lax.GatherScatterMode.PROMISE_IN_BOUNDS)

        def start(j, p):
            off = j * _C
            pltpu.async_copy(
                src_hbm.at[idx_s.at[pl.ds(off, _C)]], rows_s[p], sem_s[p])
            pltpu.async_copy(
                tgt_hbm.at[idx_t.at[pl.ds(off, _C)]], rows_t[p], sem_t[p])

        def wait(p):
            pltpu.make_async_copy(
                src_hbm.at[idx_s.at[pl.ds(0, _C)]], rows_s[p], sem_s[p]
            ).wait()
            pltpu.make_async_copy(
                tgt_hbm.at[idx_t.at[pl.ds(0, _C)]], rows_t[p], sem_t[p]
            ).wait()

        def compute(j, p):
            off = j * _C
            rs, rt = rows_s[p], rows_t[p]
            for g in range(_C // 16):

                @plsc.parallel_loop(0, 16, unroll=4,
                                    carry=jnp.zeros((16,), jnp.float32))
                def acc_loop(jj, acc, g=g):
                    e = g * 16 + jj
                    a0 = rs[e, pl.ds(0, 16)] * rt[e, pl.ds(0, 16)]
                    a1 = rs[e, pl.ds(16, 16)] * rt[e, pl.ds(16, 16)]
                    for k in range(2, _D // 16, 2):
                        a0 = a0 + (rs[e, pl.ds(16 * k, 16)]
                                   * rt[e, pl.ds(16 * k, 16)])
                        a1 = a1 + (rs[e, pl.ds(16 * (k + 1), 16)]
                                   * rt[e, pl.ds(16 * (k + 1), 16)])
                    v = a0 + a1
                    for k in range(4):
                        v = v + lane_permute(v, xors[k])
                    return jnp.where(lane == jj, v, acc)

                outbuf[pl.ds(off + g * 16, 16)] = acc_loop

        for p in range(_NBUF):
            start(p, p)

        def quad_body(k, carry):
            j0 = _NBUF * k
            for p in range(_NBUF):
                wait(p)
                compute(j0 + p, p)
                pl.when(j0 + p + _NBUF < _NCH)(
                    lambda p=p: start(j0 + p + _NBUF, p))
            return carry

        lax.fori_loop(0, _NCH // _NBUF, quad_body, 0)
        wait(0)
        compute(_NCH - 1, 0)
        pltpu.sync_copy(outbuf, out_hbm.at[pl.ds(base, _EPW)])

    return edge_dot


_sc_kernel = _make_sc_kernel()


def kernel(source_node_emb, target_node_emb, edge_label_index):
    return _sc_kernel(source_node_emb, target_node_emb,
                      edge_label_index.reshape(-1))
